# baseline (device time: 34087 ns/iter reference)
import jax
import jax.numpy as jnp
from jax import lax
from jax.experimental import pallas as pl
from jax.experimental.pallas import tpu as pltpu

N_DEV = 8
N_EXP_LOCAL = 4
N_EXP = 32


def kernel(x, router_W, route_idx, expert_W, shared_W):
    n, d = x.shape
    h = shared_W.shape[1]
    chunk = n // N_DEV

    def body(x_ref, rw_ref, idx_ref, ew_ref, sw_ref, out_ref,
             acc_ref, send_buf, recv_ref, send_sems, recv_sems):
        my = lax.axis_index("i")
        left = lax.rem(my - 1 + N_DEV, N_DEV)
        right = lax.rem(my + 1, N_DEV)

        barrier_sem = pltpu.get_barrier_semaphore()
        for nbr in (left, right):
            pl.semaphore_signal(barrier_sem, inc=1, device_id=(nbr,),
                                device_id_type=pl.DeviceIdType.MESH)
        pl.semaphore_wait(barrier_sem, 2)

        xv = x_ref[:, :]
        scores = jnp.dot(xv, rw_ref[:, :], preferred_element_type=jnp.float32)
        m = jnp.max(scores, axis=1, keepdims=True)
        p = jnp.exp(scores - m)
        probs = p / jnp.sum(p, axis=1, keepdims=True)
        idx = idx_ref[:, :]
        eids = lax.broadcasted_iota(jnp.int32, (n, N_EXP), 1)
        gate = jnp.sum(jnp.where(eids == idx, probs, 0.0), axis=1,
                       keepdims=True)

        acc = jnp.zeros((n, h), jnp.float32)
        for k in range(N_EXP_LOCAL):
            e = my * N_EXP_LOCAL + k
            w = jnp.where(idx == e, gate, 0.0)
            acc = acc + jnp.dot(xv * w, ew_ref[k],
                                preferred_element_type=jnp.float32)
        acc_ref[:, :] = acc

        for s in range(N_DEV - 1):
            c_send = lax.rem(my - s - 1 + 2 * N_DEV, N_DEV)
            send_buf[:, :] = acc_ref[pl.ds(c_send * chunk, chunk), :]
            rdma = pltpu.make_async_remote_copy(
                src_ref=send_buf,
                dst_ref=recv_ref.at[s],
                send_sem=send_sems.at[s],
                recv_sem=recv_sems.at[s],
                device_id=(right,),
                device_id_type=pl.DeviceIdType.MESH,
            )
            rdma.start()
            rdma.wait()
            c_recv = lax.rem(my - s - 2 + 2 * N_DEV, N_DEV)
            sl = pl.ds(c_recv * chunk, chunk)
            acc_ref[sl, :] = acc_ref[sl, :] + recv_ref[s, :, :]

        xs = x_ref[pl.ds(my * chunk, chunk), :]
        out_ref[:, :] = (
            jnp.dot(xs, sw_ref[:, :], preferred_element_type=jnp.float32)
            + acc_ref[pl.ds(my * chunk, chunk), :]
        )

    return pl.pallas_call(
        body,
        out_shape=jax.ShapeDtypeStruct((chunk, h), jnp.float32),
        in_specs=[pl.BlockSpec(memory_space=pltpu.VMEM)] * 5,
        out_specs=pl.BlockSpec(memory_space=pltpu.VMEM),
        scratch_shapes=[
            pltpu.VMEM((n, h), jnp.float32),
            pltpu.VMEM((chunk, h), jnp.float32),
            pltpu.VMEM((N_DEV - 1, chunk, h), jnp.float32),
            pltpu.SemaphoreType.DMA((N_DEV - 1,)),
            pltpu.SemaphoreType.DMA((N_DEV - 1,)),
        ],
        compiler_params=pltpu.CompilerParams(collective_id=0),
    )(x, router_W, route_idx, expert_W, shared_W)


# device time: 19852 ns/iter; 1.7171x vs baseline; 1.7171x over previous
import jax
import jax.numpy as jnp
from jax import lax
from jax.experimental import pallas as pl
from jax.experimental.pallas import tpu as pltpu

N_DEV = 8
N_EXP_LOCAL = 4
N_EXP = 32


def kernel(x, router_W, route_idx, expert_W, shared_W):
    n, d = x.shape
    h = shared_W.shape[1]
    chunk = n // N_DEV

    def body(x_ref, rw_ref, idx_ref, ew_ref, sw_ref, out_ref,
             acc_ref, recv_ref, send_sems, recv_sems):
        my = lax.axis_index("i")

        barrier_sem = pltpu.get_barrier_semaphore()
        for t in range(1, N_DEV):
            peer = lax.rem(my + t, N_DEV)
            pl.semaphore_signal(barrier_sem, inc=1, device_id=(peer,),
                                device_id_type=pl.DeviceIdType.MESH)
        pl.semaphore_wait(barrier_sem, N_DEV - 1)

        xv = x_ref[:, :]
        scores = jnp.dot(xv, rw_ref[:, :], preferred_element_type=jnp.float32)
        m = jnp.max(scores, axis=1, keepdims=True)
        p = jnp.exp(scores - m)
        probs = p / jnp.sum(p, axis=1, keepdims=True)
        idx = idx_ref[:, :]
        eids = lax.broadcasted_iota(jnp.int32, (n, N_EXP), 1)
        gate = jnp.sum(jnp.where(eids == idx, probs, 0.0), axis=1,
                       keepdims=True)

        acc = jnp.zeros((n, h), jnp.float32)
        for k in range(N_EXP_LOCAL):
            e = my * N_EXP_LOCAL + k
            w = jnp.where(idx == e, gate, 0.0)
            acc = acc + jnp.dot(xv * w, ew_ref[k],
                                preferred_element_type=jnp.float32)
        acc_ref[:, :] = acc

        rdmas = []
        for t in range(1, N_DEV):
            dst = lax.rem(my + t, N_DEV)
            rdma = pltpu.make_async_remote_copy(
                src_ref=acc_ref.at[pl.ds(dst * chunk, chunk)],
                dst_ref=recv_ref.at[t - 1],
                send_sem=send_sems.at[t - 1],
                recv_sem=recv_sems.at[t - 1],
                device_id=(dst,),
                device_id_type=pl.DeviceIdType.MESH,
            )
            rdma.start()
            rdmas.append(rdma)

        xs = x_ref[pl.ds(my * chunk, chunk), :]
        total = (
            jnp.dot(xs, sw_ref[:, :], preferred_element_type=jnp.float32)
            + acc_ref[pl.ds(my * chunk, chunk), :]
        )
        for t in range(1, N_DEV):
            rdmas[t - 1].wait_recv()
            total = total + recv_ref[t - 1, :, :]
        out_ref[:, :] = total

        for r in rdmas:
            r.wait_send()

    return pl.pallas_call(
        body,
        out_shape=jax.ShapeDtypeStruct((chunk, h), jnp.float32),
        in_specs=[pl.BlockSpec(memory_space=pltpu.VMEM)] * 5,
        out_specs=pl.BlockSpec(memory_space=pltpu.VMEM),
        scratch_shapes=[
            pltpu.VMEM((n, h), jnp.float32),
            pltpu.VMEM((N_DEV - 1, chunk, h), jnp.float32),
            pltpu.SemaphoreType.DMA((N_DEV - 1,)),
            pltpu.SemaphoreType.DMA((N_DEV - 1,)),
        ],
        compiler_params=pltpu.CompilerParams(collective_id=0),
    )(x, router_W, route_idx, expert_W, shared_W)


# device time: 17280 ns/iter; 1.9726x vs baseline; 1.1488x over previous
import jax
import jax.numpy as jnp
from jax import lax
from jax.experimental import pallas as pl
from jax.experimental.pallas import tpu as pltpu

N_DEV = 8
N_EXP_LOCAL = 4
N_EXP = 32


def kernel(x, router_W, route_idx, expert_W, shared_W):
    n, d = x.shape
    h = shared_W.shape[1]
    chunk = n // N_DEV

    def body(x_ref, rw_ref, idx_ref, ew_ref, sw_ref, out_ref,
             gate_ref, send_ref, recv_ref, send_sems, recv_sems):
        my = lax.axis_index("i")

        barrier_sem = pltpu.get_barrier_semaphore()
        for t in range(1, N_DEV):
            peer = lax.rem(my + t, N_DEV)
            pl.semaphore_signal(barrier_sem, inc=1, device_id=(peer,),
                                device_id_type=pl.DeviceIdType.MESH)
        pl.semaphore_wait(barrier_sem, N_DEV - 1)

        xv = x_ref[:, :]
        scores = jnp.dot(xv, rw_ref[:, :], preferred_element_type=jnp.float32)
        m = jnp.max(scores, axis=1, keepdims=True)
        p = jnp.exp(scores - m)
        probs = p / jnp.sum(p, axis=1, keepdims=True)
        idx = idx_ref[:, :]
        eids = lax.broadcasted_iota(jnp.int32, (n, N_EXP), 1)
        gate_ref[:, :] = jnp.sum(jnp.where(eids == idx, probs, 0.0), axis=1,
                                 keepdims=True)

        def partial_chunk(dst):
            rows = pl.ds(dst * chunk, chunk)
            xs = x_ref[rows, :]
            idx_c = idx_ref[rows, :]
            gate_c = gate_ref[rows, :]
            acc = jnp.zeros((chunk, h), jnp.float32)
            for k in range(N_EXP_LOCAL):
                e = my * N_EXP_LOCAL + k
                w = jnp.where(idx_c == e, gate_c, 0.0)
                acc = acc + jnp.dot(xs * w, ew_ref[k],
                                    preferred_element_type=jnp.float32)
            return acc

        rdmas = []
        for t in range(1, N_DEV):
            dst = lax.rem(my + t, N_DEV)
            send_ref[t - 1, :, :] = partial_chunk(dst).astype(jnp.bfloat16)
            rdma = pltpu.make_async_remote_copy(
                src_ref=send_ref.at[t - 1],
                dst_ref=recv_ref.at[t - 1],
                send_sem=send_sems.at[t - 1],
                recv_sem=recv_sems.at[t - 1],
                device_id=(dst,),
                device_id_type=pl.DeviceIdType.MESH,
            )
            rdma.start()
            rdmas.append(rdma)

        xs = x_ref[pl.ds(my * chunk, chunk), :]
        total = (
            jnp.dot(xs, sw_ref[:, :], preferred_element_type=jnp.float32)
            + partial_chunk(my)
        )
        for t in range(1, N_DEV):
            rdmas[t - 1].wait_recv()
            total = total + recv_ref[t - 1, :, :].astype(jnp.float32)
        out_ref[:, :] = total

        for r in rdmas:
            r.wait_send()

    return pl.pallas_call(
        body,
        out_shape=jax.ShapeDtypeStruct((chunk, h), jnp.float32),
        in_specs=[pl.BlockSpec(memory_space=pltpu.VMEM)] * 5,
        out_specs=pl.BlockSpec(memory_space=pltpu.VMEM),
        scratch_shapes=[
            pltpu.VMEM((n, 1), jnp.float32),
            pltpu.VMEM((N_DEV - 1, chunk, h), jnp.bfloat16),
            pltpu.VMEM((N_DEV - 1, chunk, h), jnp.bfloat16),
            pltpu.SemaphoreType.DMA((N_DEV - 1,)),
            pltpu.SemaphoreType.DMA((N_DEV - 1,)),
        ],
        compiler_params=pltpu.CompilerParams(collective_id=0),
    )(x, router_W, route_idx, expert_W, shared_W)
